# baseline (speedup 1.0000x reference)
import jax
import jax.numpy as jnp
from jax import lax
from jax.experimental import pallas as pl
from jax.experimental.pallas import tpu as pltpu
from jax.experimental.pallas import tpu_sc as plsc

NB, NL, NDIM = 16, 64, 8000
SRC, VBIG = 256, 8192
LANES = 16
LCHUNK = 8
TPW = 4
NT = VBIG // 128
DTILES = NDIM // 128
CTILES = 16
NCH = 4
CW = CTILES * 128
TAILC = NDIM - DTILES * 128
SHI = SRC // 128


def _sc_body(dist_hbm, pg_hbm, a5_hbm, ptr_hbm, tail_hbm, out_hbm,
             acc, dist_a, dist_b, dist_c, wv, ptr8, pg_v, tail_v,
             sem_a, sem_b, sem_c, sem_wv, sem_tail, sem_out):
    cid = lax.axis_index("c")
    sid = lax.axis_index("s")
    wid = sid * 2 + cid
    b = wid // 2
    half = wid % 2

    lane = lax.iota(jnp.int32, LANES)
    row_mask = lane < LCHUNK
    row_idx = jnp.where(row_mask, lane, 0)

    bg = (b // 8) * 8
    br = b % 8
    pltpu.sync_copy(ptr_hbm.at[pl.ds(bg, 8)], ptr8)
    pltpu.sync_copy(pg_hbm.at[pl.ds(b * NL + half * (TPW * LCHUNK), 32)],
                    pg_v.at[pl.ds(0, 32)])

    bufs = (dist_a, dist_b, dist_c, dist_a)
    sems = (sem_a, sem_b, sem_c, sem_a)

    def chunk_src(l0, c, n_t):
        return dist_hbm.at[b, pl.ds(l0, LCHUNK), pl.ds(c * CW, n_t * 128)]

    def chunk_dst(c, n_t):
        buf = bufs[c]
        if n_t == CTILES:
            return buf
        return buf.at[:, pl.ds(0, n_t * 128)]

    def n_tiles(c):
        return CTILES if c < NCH - 1 else DTILES - (NCH - 1) * CTILES

    def fire_chunks(t):
        lh = half * TPW + t
        l0 = lh * LCHUNK
        for c in range(3):
            pltpu.async_copy(chunk_src(l0, c, n_tiles(c)),
                             chunk_dst(c, n_tiles(c)), sems[c])
        pltpu.async_copy(tail_hbm.at[pl.ds(l0 * TAILC + b * NL * TAILC,
                                           LCHUNK * TAILC)], tail_v, sem_tail)

    fire_chunks(0)
    zeros = jnp.zeros((LANES,), jnp.float32)

    def task_body(t, _):
        lh = half * TPW + t
        l0 = lh * LCHUNK
        pltpu.async_copy(a5_hbm.at[b, lh], wv, sem_wv)
        pgs_vec = pg_v[pl.ds(t * LCHUNK, LANES)]
        pgs = [pgs_vec[i] for i in range(LCHUNK)]
        omp_vec = 1.0 - pgs_vec

        @pl.when(t > 0)
        def _():
            pltpu.make_async_copy(acc.at[pl.ds(0, NT // 2)],
                                  out_hbm.at[b, lh, pl.ds(0, NT // 2)],
                                  sem_out).wait()

        for c in range(NCH):
            n_t = n_tiles(c)
            buf = bufs[c]
            if c == 2:
                @pl.when(t > 0)
                def _():
                    pltpu.make_async_copy(
                        acc.at[pl.ds(NT // 2, NT // 2)],
                        out_hbm.at[b, lh, pl.ds(NT // 2, NT // 2)],
                        sem_out).wait()
            pltpu.make_async_copy(chunk_src(l0, c, n_t), chunk_dst(c, n_t),
                                  sems[c]).wait()

            def scale_c(tt, _, c=c, buf=buf, pgs=pgs):
                for i in range(LCHUNK):
                    for p in range(8):
                        sl = pl.ds(p * LANES, LANES)
                        acc[c * CTILES + tt, i, sl] = (
                            buf[i, pl.ds(tt * 128 + p * LANES, LANES)]
                            * pgs[i])
                return 0

            lax.fori_loop(0, n_t, scale_c, 0)
            if c == 0:
                pltpu.async_copy(chunk_src(l0, 3, n_tiles(3)),
                                 chunk_dst(3, n_tiles(3)), sems[3])

        pltpu.make_async_copy(tail_hbm.at[pl.ds(0, LCHUNK * TAILC)], tail_v,
                              sem_tail).wait()
        for i in range(LCHUNK):
            for p in range(TAILC // LANES):
                acc[DTILES, i, pl.ds(p * LANES, LANES)] = (
                    tail_v[pl.ds(i * TAILC + p * LANES, LANES)] * pgs[i])
            for p in range(TAILC // LANES, 8):
                acc[DTILES, i, pl.ds(p * LANES, LANES)] = zeros
            for p in range(8):
                acc[NT - 1, i, pl.ds(p * LANES, LANES)] = zeros

        @pl.when(t + 1 < TPW)
        def _():
            lnext = (half * TPW + t + 1) * LCHUNK
            for c in range(3):
                pltpu.async_copy(
                    dist_hbm.at[b, pl.ds(lnext, LCHUNK),
                                pl.ds(c * CW, n_tiles(c) * 128)],
                    chunk_dst(c, n_tiles(c)), sems[c])
            pltpu.async_copy(tail_hbm.at[pl.ds(lnext * TAILC
                                               + b * NL * TAILC,
                                               LCHUNK * TAILC)],
                             tail_v, sem_tail)

        pltpu.make_async_copy(a5_hbm.at[b, lh], wv, sem_wv).wait()

        def scat_group(g, _):
            pvec = ptr8[br, pl.ds(g * LANES, LANES)]
            phi = lax.shift_right_logical(pvec, 7)
            plo = lax.bitwise_and(pvec, 127)
            for k in range(LANES):
                s = g * LANES + k
                shi = jnp.full((LANES,), lax.shift_right_logical(s, 7),
                               jnp.int32)
                slo = jnp.full((LANES,), lax.bitwise_and(s, 127), jnp.int32)
                aval = plsc.load_gather(wv, [shi, row_idx, slo],
                                        mask=row_mask)
                tvec = jnp.full((LANES,), phi[k], jnp.int32)
                cvec = jnp.full((LANES,), plo[k], jnp.int32)
                plsc.addupdate_scatter(acc, [tvec, row_idx, cvec],
                                       aval * omp_vec, mask=row_mask)
            return 0

        lax.fori_loop(0, SRC // LANES, scat_group, 0)

        pltpu.async_copy(acc.at[pl.ds(0, NT // 2)],
                         out_hbm.at[b, lh, pl.ds(0, NT // 2)], sem_out)
        pltpu.async_copy(acc.at[pl.ds(NT // 2, NT // 2)],
                         out_hbm.at[b, lh, pl.ds(NT // 2, NT // 2)], sem_out)
        return 0

    lax.fori_loop(0, TPW, task_body, 0)
    pltpu.make_async_copy(acc, out_hbm.at[b, half * TPW + TPW - 1],
                          sem_out).wait()


@jax.jit
def _copy_net(dist_t, pg1, a5, ptr1, tail1):
    mesh = plsc.VectorSubcoreMesh(core_axis_name="c", subcore_axis_name="s")
    f = pl.kernel(
        _sc_body,
        out_type=jax.ShapeDtypeStruct((NB, NL // LCHUNK, NT, LCHUNK, 128),
                                      jnp.float32),
        mesh=mesh,
        scratch_types=[
            pltpu.VMEM((NT, LCHUNK, 128), jnp.float32),
            pltpu.VMEM((LCHUNK, CW), jnp.float32),
            pltpu.VMEM((LCHUNK, CW), jnp.float32),
            pltpu.VMEM((LCHUNK, CW), jnp.float32),
            pltpu.VMEM((SHI, LCHUNK, 128), jnp.float32),
            pltpu.VMEM((8, SRC), jnp.int32),
            pltpu.VMEM((48,), jnp.float32),
            pltpu.VMEM((LCHUNK * TAILC,), jnp.float32),
            pltpu.SemaphoreType.DMA,
            pltpu.SemaphoreType.DMA,
            pltpu.SemaphoreType.DMA,
            pltpu.SemaphoreType.DMA,
            pltpu.SemaphoreType.DMA,
            pltpu.SemaphoreType.DMA,
        ],
        compiler_params=pltpu.CompilerParams(use_tc_tiling_on_sc=True,
                                             needs_layout_passes=False,
                                             disable_bounds_checks=True,
                                             skip_device_barrier=True),
    )
    out5 = f(dist_t, pg1, a5, ptr1, tail1)
    return out5.transpose(0, 1, 3, 2, 4).reshape(NB, NL, VBIG)


def kernel(dist_t, p_gen, alph_t, batch_vocab, pointer):
    del batch_vocab
    pg1 = p_gen.reshape(-1)
    ptr1 = pointer.astype(jnp.int32)
    tail1 = dist_t[:, :, DTILES * 128:].reshape(-1)
    a5 = alph_t.reshape(NB, SHI, 128, NL // LCHUNK, LCHUNK).transpose(
        0, 3, 1, 4, 2)
    return _copy_net(dist_t, pg1, a5, ptr1, tail1)


# speedup vs baseline: 1.0074x; 1.0074x over previous
import jax
import jax.numpy as jnp
from jax import lax
from jax.experimental import pallas as pl
from jax.experimental.pallas import tpu as pltpu
from jax.experimental.pallas import tpu_sc as plsc

NB, NL, NDIM = 16, 64, 8000
SRC, VBIG = 256, 8192
LANES = 16
LCHUNK = 8
TPW = 4
NT = VBIG // 128
DTILES = NDIM // 128
CTILES = 16
NCH = 4
CW = CTILES * 128
TAILC = NDIM - DTILES * 128
SHI = SRC // 128


def _sc_body(dist_hbm, pg_hbm, a5_hbm, ptr_hbm, tail_hbm, out_hbm,
             acc, dist_a, dist_b, dist_c, wv, ptr8, pg_v, tail_v,
             sem_a, sem_b, sem_c, sem_wv, sem_tail, sem_out):
    cid = lax.axis_index("c")
    sid = lax.axis_index("s")
    wid = sid * 2 + cid
    b = wid // 2
    half = wid % 2

    lane = lax.iota(jnp.int32, LANES)
    row_mask = lane < LCHUNK
    row_idx = jnp.where(row_mask, lane, 0)

    bufs = (dist_a, dist_b, dist_c, dist_a)
    sems = (sem_a, sem_b, sem_c, sem_a)

    def chunk_src(l0, c, n_t):
        return dist_hbm.at[b, pl.ds(l0, LCHUNK), pl.ds(c * CW, n_t * 128)]

    def chunk_dst(c, n_t):
        buf = bufs[c]
        if n_t == CTILES:
            return buf
        return buf.at[:, pl.ds(0, n_t * 128)]

    def n_tiles(c):
        return CTILES if c < NCH - 1 else DTILES - (NCH - 1) * CTILES

    def fire_chunks(t):
        lh = half * TPW + t
        l0 = lh * LCHUNK
        for c in range(3):
            pltpu.async_copy(chunk_src(l0, c, n_tiles(c)),
                             chunk_dst(c, n_tiles(c)), sems[c])
        pltpu.async_copy(tail_hbm.at[pl.ds(l0 * TAILC + b * NL * TAILC,
                                           LCHUNK * TAILC)], tail_v, sem_tail)

    fire_chunks(0)
    bg = (b // 8) * 8
    br = b % 8
    pltpu.sync_copy(ptr_hbm.at[pl.ds(bg, 8)], ptr8)
    pltpu.sync_copy(pg_hbm.at[pl.ds(b * NL + half * (TPW * LCHUNK), 32)],
                    pg_v.at[pl.ds(0, 32)])
    zeros = jnp.zeros((LANES,), jnp.float32)

    def task_body(t, _):
        lh = half * TPW + t
        l0 = lh * LCHUNK
        pltpu.async_copy(a5_hbm.at[b, lh], wv, sem_wv)
        pgs_vec = pg_v[pl.ds(t * LCHUNK, LANES)]
        pgs = [pgs_vec[i] for i in range(LCHUNK)]
        omp_vec = 1.0 - pgs_vec

        @pl.when(t > 0)
        def _():
            pltpu.make_async_copy(acc.at[pl.ds(0, NT // 2)],
                                  out_hbm.at[b, lh, pl.ds(0, NT // 2)],
                                  sem_out).wait()

        for c in range(NCH):
            n_t = n_tiles(c)
            buf = bufs[c]
            if c == 2:
                @pl.when(t > 0)
                def _():
                    pltpu.make_async_copy(
                        acc.at[pl.ds(NT // 2, NT // 2)],
                        out_hbm.at[b, lh, pl.ds(NT // 2, NT // 2)],
                        sem_out).wait()
            pltpu.make_async_copy(chunk_src(l0, c, n_t), chunk_dst(c, n_t),
                                  sems[c]).wait()

            def scale_c(tt, _, c=c, buf=buf, pgs=pgs):
                for i in range(LCHUNK):
                    for p in range(8):
                        sl = pl.ds(p * LANES, LANES)
                        acc[c * CTILES + tt, i, sl] = (
                            buf[i, pl.ds(tt * 128 + p * LANES, LANES)]
                            * pgs[i])
                return 0

            lax.fori_loop(0, n_t, scale_c, 0)
            if c == 0:
                pltpu.async_copy(chunk_src(l0, 3, n_tiles(3)),
                                 chunk_dst(3, n_tiles(3)), sems[3])

        pltpu.make_async_copy(tail_hbm.at[pl.ds(0, LCHUNK * TAILC)], tail_v,
                              sem_tail).wait()
        for i in range(LCHUNK):
            for p in range(TAILC // LANES):
                acc[DTILES, i, pl.ds(p * LANES, LANES)] = (
                    tail_v[pl.ds(i * TAILC + p * LANES, LANES)] * pgs[i])
            for p in range(TAILC // LANES, 8):
                acc[DTILES, i, pl.ds(p * LANES, LANES)] = zeros
            for p in range(8):
                acc[NT - 1, i, pl.ds(p * LANES, LANES)] = zeros

        @pl.when(t + 1 < TPW)
        def _():
            lnext = (half * TPW + t + 1) * LCHUNK
            for c in range(3):
                pltpu.async_copy(
                    dist_hbm.at[b, pl.ds(lnext, LCHUNK),
                                pl.ds(c * CW, n_tiles(c) * 128)],
                    chunk_dst(c, n_tiles(c)), sems[c])
            pltpu.async_copy(tail_hbm.at[pl.ds(lnext * TAILC
                                               + b * NL * TAILC,
                                               LCHUNK * TAILC)],
                             tail_v, sem_tail)

        pltpu.make_async_copy(a5_hbm.at[b, lh], wv, sem_wv).wait()

        def scat_group(g, _):
            pvec = ptr8[br, pl.ds(g * LANES, LANES)]
            phi = lax.shift_right_logical(pvec, 7)
            plo = lax.bitwise_and(pvec, 127)
            for k in range(LANES):
                s = g * LANES + k
                shi = jnp.full((LANES,), lax.shift_right_logical(s, 7),
                               jnp.int32)
                slo = jnp.full((LANES,), lax.bitwise_and(s, 127), jnp.int32)
                aval = plsc.load_gather(wv, [shi, row_idx, slo],
                                        mask=row_mask)
                tvec = jnp.full((LANES,), phi[k], jnp.int32)
                cvec = jnp.full((LANES,), plo[k], jnp.int32)
                plsc.addupdate_scatter(acc, [tvec, row_idx, cvec],
                                       aval * omp_vec, mask=row_mask)
            return 0

        lax.fori_loop(0, SRC // LANES, scat_group, 0)

        pltpu.async_copy(acc.at[pl.ds(0, NT // 2)],
                         out_hbm.at[b, lh, pl.ds(0, NT // 2)], sem_out)
        pltpu.async_copy(acc.at[pl.ds(NT // 2, NT // 2)],
                         out_hbm.at[b, lh, pl.ds(NT // 2, NT // 2)], sem_out)
        return 0

    lax.fori_loop(0, TPW, task_body, 0)
    pltpu.make_async_copy(acc, out_hbm.at[b, half * TPW + TPW - 1],
                          sem_out).wait()


@jax.jit
def _copy_net(dist_t, pg1, a5, ptr1, tail1):
    mesh = plsc.VectorSubcoreMesh(core_axis_name="c", subcore_axis_name="s")
    f = pl.kernel(
        _sc_body,
        out_type=jax.ShapeDtypeStruct((NB, NL // LCHUNK, NT, LCHUNK, 128),
                                      jnp.float32),
        mesh=mesh,
        scratch_types=[
            pltpu.VMEM((NT, LCHUNK, 128), jnp.float32),
            pltpu.VMEM((LCHUNK, CW), jnp.float32),
            pltpu.VMEM((LCHUNK, CW), jnp.float32),
            pltpu.VMEM((LCHUNK, CW), jnp.float32),
            pltpu.VMEM((SHI, LCHUNK, 128), jnp.float32),
            pltpu.VMEM((8, SRC), jnp.int32),
            pltpu.VMEM((48,), jnp.float32),
            pltpu.VMEM((LCHUNK * TAILC,), jnp.float32),
            pltpu.SemaphoreType.DMA,
            pltpu.SemaphoreType.DMA,
            pltpu.SemaphoreType.DMA,
            pltpu.SemaphoreType.DMA,
            pltpu.SemaphoreType.DMA,
            pltpu.SemaphoreType.DMA,
        ],
        compiler_params=pltpu.CompilerParams(use_tc_tiling_on_sc=True,
                                             needs_layout_passes=False),
    )
    out5 = f(dist_t, pg1, a5, ptr1, tail1)
    return out5.transpose(0, 1, 3, 2, 4).reshape(NB, NL, VBIG)


def kernel(dist_t, p_gen, alph_t, batch_vocab, pointer):
    del batch_vocab
    pg1 = p_gen.reshape(-1)
    ptr1 = pointer.astype(jnp.int32)
    tail1 = dist_t[:, :, DTILES * 128:].reshape(-1)
    a5 = alph_t.reshape(NB, SHI, 128, NL // LCHUNK, LCHUNK).transpose(
        0, 3, 1, 4, 2)
    return _copy_net(dist_t, pg1, a5, ptr1, tail1)
